# baseline (speedup 1.0000x reference)
import functools

import jax
import jax.numpy as jnp
from jax import lax
from jax.experimental import pallas as pl
from jax.experimental.pallas import tpu as pltpu
from jax.experimental.pallas import tpu_sc as plsc

K = 150
STR = 161
TAB = 24160
SSTR = 150
STAB = 22504
NW = 32
WPB = 4
KC = 15
HB = 8


def _make_sc_kernel(Bc, Hc, Wc):
    npix = Bc * Hc * Wc
    ppw = npix // NW
    hpw = ppw // Wc
    nhb = hpw // HB
    nkb = K // KC
    ngroups = Wc // 16
    ngq = HB * ngroups
    cnt = KC * HB * Wc * 4
    mesh = plsc.VectorSubcoreMesh(core_axis_name="c", subcore_axis_name="s")

    @functools.partial(
        pl.kernel,
        mesh=mesh,
        compiler_params=pltpu.CompilerParams(needs_layout_passes=False),
        out_type=jax.ShapeDtypeStruct((Bc, K, Hc, Wc), jnp.float32),
        scratch_types=[
            pltpu.VMEM((TAB,), jnp.float32),
            pltpu.VMEM((TAB,), jnp.float32),
            pltpu.VMEM((STAB,), jnp.float32),
            pltpu.VMEM((ppw,), jnp.int32),
            pltpu.VMEM((ppw,), jnp.int32),
            pltpu.VMEM((ngq, 16), jnp.float32),
            pltpu.VMEM((KC, HB, Wc), jnp.float32),
            pltpu.VMEM((KC, HB, Wc), jnp.float32),
            pltpu.SemaphoreType.DMA,
            pltpu.SemaphoreType.DMA,
        ],
    )
    def sc_kernel(a_hbm, b_hbm, s_hbm, xt_hbm, x0_hbm, out_hbm,
                  a_v, b_v, s_v, xt_v, x0_v, inv_v, ob0, ob1, sem0, sem1):
        cid = lax.axis_index("c")
        sid = lax.axis_index("s")
        wid = sid * 2 + cid
        batch = wid // WPB
        h0 = (wid % WPB) * hpw
        row0 = wid * ppw
        pltpu.sync_copy(a_hbm.at[pl.ds(batch * TAB, TAB)], a_v)
        pltpu.sync_copy(b_hbm.at[pl.ds(batch * TAB, TAB)], b_v)
        pltpu.sync_copy(s_hbm.at[pl.ds(batch * STAB, STAB)], s_v)
        pltpu.sync_copy(xt_hbm.at[pl.ds(row0, ppw)], xt_v)
        pltpu.sync_copy(x0_hbm.at[pl.ds(row0, ppw)], x0_v)

        def do_chunk(kb, hb, obuf, sem):
            k0 = kb * KC
            pblk = hb * (HB * Wc)
            dst = out_hbm.at[batch, pl.ds(k0, KC), pl.ds(h0 + hb * HB, HB), :]
            ci = hb * nkb + kb

            @pl.when(ci >= 2)
            def _wait_prev():
                pltpu.make_async_copy(obuf, dst, sem).wait()

            def do_row(hr, carry2):
                prow = pblk + hr * Wc
                xts, x0s, invs = [], [], []
                for g in range(ngroups):
                    xts.append(xt_v[pl.ds(prow + g * 16, 16)])
                    x0s.append(x0_v[pl.ds(prow + g * 16, 16)])
                    invs.append(inv_v[hr * ngroups + g, pl.ds(0, 16)])

                @plsc.parallel_loop(0, KC, unroll=1)
                def _do_cat(cl):
                    coff = (k0 + cl) * STR
                    for g in range(ngroups):
                        av = plsc.load_gather(a_v, [xts[g] + coff])
                        bv = plsc.load_gather(b_v, [x0s[g] + coff])
                        obuf[cl, hr, pl.ds(g * 16, 16)] = av * bv * invs[g]

                return carry2

            lax.fori_loop(0, HB, do_row, 0)
            pltpu.async_copy(obuf, dst, sem)

        def hblock_body(hb, carry):
            pblk = hb * (HB * Wc)

            def build_inv(q, carry2):
                xt_vec = xt_v[pl.ds(pblk + q * 16, 16)]
                x0_vec = x0_v[pl.ds(pblk + q * 16, 16)]
                den = plsc.load_gather(s_v, [xt_vec * SSTR + x0_vec])
                inv_v[q, pl.ds(0, 16)] = 1.0 / (den + 1e-10)
                return carry2

            lax.fori_loop(0, ngq, build_inv, 0, unroll=4)

            def kb_pair(p, carry2):
                do_chunk(2 * p, hb, ob0, sem0)
                do_chunk(2 * p + 1, hb, ob1, sem1)
                return carry2

            lax.fori_loop(0, nkb // 2, kb_pair, 0)
            return carry

        lax.fori_loop(0, nhb, hblock_body, 0)
        last = out_hbm.at[batch, pl.ds(0, KC), pl.ds(h0, HB), :]
        pltpu.make_async_copy(ob0, last, sem0).wait()
        pltpu.make_async_copy(ob1, last, sem1).wait()

    return sc_kernel


def kernel(x_0, x_t, t, Q_t, Q_bar):
    Bc, Hc, Wc = x_0.shape
    Kc = Q_t.shape[-1]
    npix = Bc * Hc * Wc
    tt = t.astype(jnp.int32)
    Q_t, Q_bar = lax.optimization_barrier((Q_t, Q_bar))
    Qt_sel = Q_t[tt]
    tm1 = jnp.clip(tt - 1, 0, None)
    Qb_sel = Q_bar[tm1]
    eye = jnp.eye(Kc, dtype=jnp.float32)
    is0 = (tt == 0)[:, None, None]
    Qb_sel = jnp.where(is0, eye[None], Qb_sel)
    s_tab = jnp.einsum("bki,bkj->bij", Qt_sel, Qb_sel)

    def flatten(tabs, stride, total):
        padded = jnp.pad(tabs, ((0, 0), (0, 0), (0, stride - Kc)))
        flat = padded.reshape(Bc, Kc * stride)
        flat = jnp.pad(flat, ((0, 0), (0, total - Kc * stride)))
        return flat.reshape(Bc * total)

    a_tab = flatten(Qt_sel, STR, TAB)
    b_tab = flatten(Qb_sel, STR, TAB)
    s_flat = flatten(s_tab, SSTR, STAB)
    xt_flat = x_t.reshape(npix).astype(jnp.int32)
    x0_flat = x_0.reshape(npix).astype(jnp.int32)
    out = _make_sc_kernel(Bc, Hc, Wc)(a_tab, b_tab, s_flat, xt_flat, x0_flat)
    out = lax.optimization_barrier(out)
    return jnp.transpose(out, (0, 2, 3, 1))


# speedup vs baseline: 1.0376x; 1.0376x over previous
import functools

import jax
import jax.numpy as jnp
from jax import lax
from jax.experimental import pallas as pl
from jax.experimental.pallas import tpu as pltpu
from jax.experimental.pallas import tpu_sc as plsc

K = 150
STR = 161
TAB = 24160
SSTR = 150
STAB = 22504
NW = 32
WPB = 4
KC = 15
HB = 8


def _make_sc_kernel(Bc, Hc, Wc):
    npix = Bc * Hc * Wc
    ppw = npix // NW
    hpw = ppw // Wc
    nhb = hpw // HB
    nkb = K // KC
    ngroups = Wc // 16
    ngq = HB * ngroups
    cnt = KC * HB * Wc * 4
    mesh = plsc.VectorSubcoreMesh(core_axis_name="c", subcore_axis_name="s")

    @functools.partial(
        pl.kernel,
        mesh=mesh,
        compiler_params=pltpu.CompilerParams(needs_layout_passes=False),
        out_type=jax.ShapeDtypeStruct((Bc, K, Hc, Wc), jnp.float32),
        scratch_types=[
            pltpu.VMEM((TAB,), jnp.float32),
            pltpu.VMEM((TAB,), jnp.float32),
            pltpu.VMEM((STAB,), jnp.float32),
            pltpu.VMEM((ppw,), jnp.int32),
            pltpu.VMEM((ppw,), jnp.int32),
            pltpu.VMEM((ngq, 16), jnp.float32),
            pltpu.VMEM((KC, HB, Wc), jnp.float32),
            pltpu.VMEM((KC, HB, Wc), jnp.float32),
            pltpu.SemaphoreType.DMA,
            pltpu.SemaphoreType.DMA,
        ],
    )
    def sc_kernel(a_hbm, b_hbm, s_hbm, xt_hbm, x0_hbm, out_hbm,
                  a_v, b_v, s_v, xt_v, x0_v, inv_v, ob0, ob1, sem0, sem1):
        cid = lax.axis_index("c")
        sid = lax.axis_index("s")
        wid = sid * 2 + cid
        batch = wid // WPB
        h0 = (wid % WPB) * hpw
        row0 = wid * ppw
        pltpu.sync_copy(a_hbm.at[pl.ds(batch * TAB, TAB)], a_v)
        pltpu.sync_copy(b_hbm.at[pl.ds(batch * TAB, TAB)], b_v)
        pltpu.sync_copy(s_hbm.at[pl.ds(batch * STAB, STAB)], s_v)
        pltpu.sync_copy(xt_hbm.at[pl.ds(row0, ppw)], xt_v)
        pltpu.sync_copy(x0_hbm.at[pl.ds(row0, ppw)], x0_v)

        def do_chunk(kb, hb, obuf, sem):
            k0 = kb * KC
            pblk = hb * (HB * Wc)
            dst = out_hbm.at[batch, pl.ds(k0, KC), pl.ds(h0 + hb * HB, HB), :]
            ci = hb * nkb + kb

            @pl.when(ci >= 2)
            def _wait_prev():
                pltpu.make_async_copy(obuf, dst, sem).wait()

            def do_row(hr, carry2):
                prow = pblk + hr * Wc
                xts, x0s, invs = [], [], []
                for g in range(ngroups):
                    xts.append(xt_v[pl.ds(prow + g * 16, 16)])
                    x0s.append(x0_v[pl.ds(prow + g * 16, 16)])
                    invs.append(inv_v[hr * ngroups + g, pl.ds(0, 16)])

                @plsc.parallel_loop(0, KC, unroll=1)
                def _do_cat(cl):
                    coff = (k0 + cl) * STR
                    for g in range(ngroups):
                        av = plsc.load_gather(a_v, [xts[g] + coff])
                        bv = plsc.load_gather(b_v, [x0s[g] + coff])
                        obuf[cl, hr, pl.ds(g * 16, 16)] = av * bv * invs[g]

                return carry2

            lax.fori_loop(0, HB, do_row, 0)
            pltpu.async_copy(obuf, dst, sem)

        def hblock_body(hb, carry):
            pblk = hb * (HB * Wc)

            def build_inv(q, carry2):
                xt_vec = xt_v[pl.ds(pblk + q * 16, 16)]
                x0_vec = x0_v[pl.ds(pblk + q * 16, 16)]
                den = plsc.load_gather(s_v, [xt_vec * SSTR + x0_vec])
                inv_v[q, pl.ds(0, 16)] = 1.0 / (den + 1e-10)
                return carry2

            lax.fori_loop(0, ngq, build_inv, 0, unroll=4)

            def kb_pair(p, carry2):
                do_chunk(2 * p, hb, ob0, sem0)
                do_chunk(2 * p + 1, hb, ob1, sem1)
                return carry2

            lax.fori_loop(0, nkb // 2, kb_pair, 0)
            return carry

        lax.fori_loop(0, nhb, hblock_body, 0)
        last = out_hbm.at[batch, pl.ds(0, KC), pl.ds(h0, HB), :]
        pltpu.make_async_copy(ob0, last, sem0).wait()
        pltpu.make_async_copy(ob1, last, sem1).wait()

    return sc_kernel


def kernel(x_0, x_t, t, Q_t, Q_bar):
    Bc, Hc, Wc = x_0.shape
    Kc = Q_t.shape[-1]
    npix = Bc * Hc * Wc
    tt = t.astype(jnp.int32)
    Qt_sel = Q_t[tt]
    tm1 = jnp.clip(tt - 1, 0, None)
    Qb_sel = Q_bar[tm1]
    eye = jnp.eye(Kc, dtype=jnp.float32)
    is0 = (tt == 0)[:, None, None]
    Qb_sel = jnp.where(is0, eye[None], Qb_sel)
    s_tab = jnp.einsum("bki,bkj->bij", Qt_sel, Qb_sel)

    def flatten(tabs, stride, total):
        padded = jnp.pad(tabs, ((0, 0), (0, 0), (0, stride - Kc)))
        flat = padded.reshape(Bc, Kc * stride)
        flat = jnp.pad(flat, ((0, 0), (0, total - Kc * stride)))
        return flat.reshape(Bc * total)

    a_tab = flatten(Qt_sel, STR, TAB)
    b_tab = flatten(Qb_sel, STR, TAB)
    s_flat = flatten(s_tab, SSTR, STAB)
    xt_flat = x_t.reshape(npix).astype(jnp.int32)
    x0_flat = x_0.reshape(npix).astype(jnp.int32)
    out = _make_sc_kernel(Bc, Hc, Wc)(a_tab, b_tab, s_flat, xt_flat, x0_flat)
    out = lax.optimization_barrier(out)
    return jnp.transpose(out, (0, 2, 3, 1))
